# Initial kernel scaffold; baseline (speedup 1.0000x reference)
import jax
import jax.numpy as jnp
from jax.experimental import pallas as pl


def kernel(xyz, features, W1, b1, W2, b2, W3, b3):
    raise NotImplementedError("write your pallas kernel here")


# speedup vs baseline: 16.5211x; 16.5211x over previous
import functools

import numpy as np
import jax
import jax.numpy as jnp
from jax import lax
from jax.experimental import pallas as pl
from jax.experimental.pallas import tpu as pltpu
from jax.experimental.pallas import tpu_sc as plsc

_B, _N, _C = 4, 8192, 16
_P, _S = 1024, 32
_R2 = np.float32(0.8 * 0.8)
_PCH = 128
_NJ = _P // _PCH


def _fps_body(x_ref, y_ref, z_ref, nx_ref, ny_ref, nz_ref):
    x = x_ref[...]
    y = y_ref[...]
    z = z_ref[...]
    lane = lax.broadcasted_iota(jnp.int32, (_B, _N), 1)
    lx = x[:, 0:1]
    ly = y[:, 0:1]
    lz = z[:, 0:1]
    pcol = lax.broadcasted_iota(jnp.int32, (_B, _P), 1)
    nx0 = jnp.broadcast_to(lx, (_B, _P))
    ny0 = jnp.broadcast_to(ly, (_B, _P))
    nz0 = jnp.broadcast_to(lz, (_B, _P))
    dists0 = jnp.full((_B, _N), 1e10, dtype=jnp.float32)

    def body(i, carry):
        dists, lx, ly, lz, nxa, nya, nza = carry
        dx = x - lx
        dy = y - ly
        dz = z - lz
        d = dx * dx + dy * dy + dz * dz
        dists = jnp.minimum(dists, d)
        m = jnp.max(dists, axis=-1, keepdims=True)
        cand = jnp.where(dists == m, lane, _N)
        nxt = jnp.min(cand, axis=-1, keepdims=True)
        sel = lane == nxt
        lx = jnp.sum(jnp.where(sel, x, 0.0), axis=-1, keepdims=True)
        ly = jnp.sum(jnp.where(sel, y, 0.0), axis=-1, keepdims=True)
        lz = jnp.sum(jnp.where(sel, z, 0.0), axis=-1, keepdims=True)
        slot = pcol == i
        nxa = jnp.where(slot, lx, nxa)
        nya = jnp.where(slot, ly, nya)
        nza = jnp.where(slot, lz, nza)
        return dists, lx, ly, lz, nxa, nya, nza

    carry = lax.fori_loop(1, _P, body, (dists0, lx, ly, lz, nx0, ny0, nz0))
    nx_ref[...] = carry[4]
    ny_ref[...] = carry[5]
    nz_ref[...] = carry[6]


def _fps(xs, ys, zs):
    out = jax.ShapeDtypeStruct((_B, _P), jnp.float32)
    return pl.pallas_call(
        _fps_body,
        out_shape=(out, out, out),
    )(xs, ys, zs)


def _bq_body(xc_ref, yc_ref, zc_ref, cx_ref, cy_ref, cz_ref, idx_ref, cnt_ref):
    b = pl.program_id(0)
    xc = xc_ref[0]
    yc = yc_ref[0]
    zc = zc_ref[0]
    cx = cx_ref[0, 0]
    cy = cy_ref[0, 0]
    cz = cz_ref[0, 0]
    dx = xc - cx
    dy = yc - cy
    dz = zc - cz
    d2 = dx * dx + dy * dy + dz * dz
    within = d2 < _R2
    pid = lax.broadcasted_iota(jnp.int32, (_N, _PCH), 0)
    keys = jnp.where(within, pid, _N)
    cnt_ref[0, 0] = jnp.sum(within.astype(jnp.int32), axis=0, keepdims=True)
    cols = []
    for _ in range(_S):
        m = jnp.min(keys, axis=0, keepdims=True)
        cols.append(m)
        keys = jnp.where(keys == m, _N, keys)
    idx = jnp.concatenate(cols, axis=0)
    first = jnp.where(cols[0] == _N, 0, cols[0])
    idx = jnp.where(idx == _N, first, idx)
    idx_ref[0, 0] = idx + b * _N


def _ballquery(xc, yc, zc, cx, cy, cz):
    pt_spec = pl.BlockSpec((1, _N, 1), lambda b, j: (b, 0, 0))
    cen_spec = pl.BlockSpec((1, 1, 1, _PCH), lambda b, j: (b, j, 0, 0))
    return pl.pallas_call(
        _bq_body,
        grid=(_B, _NJ),
        in_specs=[pt_spec, pt_spec, pt_spec, cen_spec, cen_spec, cen_spec],
        out_specs=[
            pl.BlockSpec((1, 1, _S, _PCH), lambda b, j: (b, j, 0, 0)),
            pl.BlockSpec((1, 1, 1, _PCH), lambda b, j: (b, j, 0, 0)),
        ],
        out_shape=[
            jax.ShapeDtypeStruct((_B, _NJ, _S, _PCH), jnp.int32),
            jax.ShapeDtypeStruct((_B, _NJ, 1, _PCH), jnp.int32),
        ],
    )(xc, yc, zc, cx, cy, cz)


_NW = 32
_ROWS = _B * _S * _P
_RPW = _ROWS // _NW
_GCH = 128
_HALF = 2048


def _gather(table, idxflat):
    mesh = plsc.VectorSubcoreMesh(core_axis_name="c", subcore_axis_name="s")

    @functools.partial(
        pl.kernel,
        out_type=jax.ShapeDtypeStruct((_ROWS, 32), jnp.float32),
        mesh=mesh,
        compiler_params=pltpu.CompilerParams(use_tc_tiling_on_sc=False),
        scratch_types=[
            pltpu.VMEM((_RPW,), jnp.int32),
            pltpu.VMEM((_HALF, 32), jnp.float32),
            pltpu.SemaphoreType.DMA,
        ],
    )
    def k(table_hbm, idx_hbm, out_hbm, idx_v, rows_v, sem):
        wid = lax.axis_index("s") * 2 + lax.axis_index("c")
        base = wid * _RPW
        pltpu.sync_copy(idx_hbm.at[pl.ds(base, _RPW)], idx_v)
        for h in range(_RPW // _HALF):
            copies = []
            for c in range(_HALF // _GCH):
                off = h * _HALF + c * _GCH
                copies.append(pltpu.async_copy(
                    table_hbm.at[idx_v.at[pl.ds(off, _GCH)]],
                    rows_v.at[pl.ds(c * _GCH, _GCH)], sem))
            for cp in copies:
                cp.wait()
            pltpu.sync_copy(rows_v, out_hbm.at[pl.ds(base + h * _HALF, _HALF)])

    return k(table, idxflat)


def _mlp_body(g_ref, nx_ref, ny_ref, nz_ref, cnt_ref,
              w1_ref, b1_ref, w2_ref, b2_ref, w3_ref, b3_ref, out_ref):
    g = g_ref[0]
    nx = nx_ref[0]
    ny = ny_ref[0]
    nz = nz_ref[0]
    cen = jnp.concatenate(
        [nx[:, :, None], ny[:, :, None], nz[:, :, None],
         jnp.zeros((1, _P, 29), jnp.float32)], axis=2)
    x = (g.reshape(_S, _P, 32) - cen).reshape(_S * _P, 32)
    h = jnp.maximum(
        jnp.dot(x, w1_ref[...], preferred_element_type=jnp.float32)
        + b1_ref[...], 0.0)
    h = jnp.maximum(
        jnp.dot(h, w2_ref[...], preferred_element_type=jnp.float32)
        + b2_ref[...], 0.0)
    h = jnp.maximum(
        jnp.dot(h, w3_ref[...], preferred_element_type=jnp.float32)
        + b3_ref[...], 0.0)
    red = jnp.max(h.reshape(_S, _P, 64), axis=0)
    mask = cnt_ref[0] > 0
    out_ref[0] = jnp.where(mask, red, 0.0)


def _mlp_pool(g, nx, ny, nz, cnt, w1p, b1, w2t, b2, w3t, b3):
    full = lambda shape: pl.BlockSpec(shape, lambda b: (0,) * len(shape))
    return pl.pallas_call(
        _mlp_body,
        grid=(_B,),
        in_specs=[
            pl.BlockSpec((1, _S * _P, 32), lambda b: (b, 0, 0)),
            pl.BlockSpec((1, 1, _P), lambda b: (b, 0, 0)),
            pl.BlockSpec((1, 1, _P), lambda b: (b, 0, 0)),
            pl.BlockSpec((1, 1, _P), lambda b: (b, 0, 0)),
            pl.BlockSpec((1, _P, 1), lambda b: (b, 0, 0)),
            full((32, 32)), full((1, 32)),
            full((32, 32)), full((1, 32)),
            full((32, 64)), full((1, 64)),
        ],
        out_specs=pl.BlockSpec((1, _P, 64), lambda b: (b, 0, 0)),
        out_shape=jax.ShapeDtypeStruct((_B, _P, 64), jnp.float32),
    )(g, nx, ny, nz, cnt, w1p, b1, w2t, b2, w3t, b3)


def kernel(xyz, features, W1, b1, W2, b2, W3, b3):
    xs = xyz[:, :, 0]
    ys = xyz[:, :, 1]
    zs = xyz[:, :, 2]

    nx, ny, nz = _fps(xs, ys, zs)

    idxg, cntg = _ballquery(
        xs[:, :, None], ys[:, :, None], zs[:, :, None],
        nx.reshape(_B, _NJ, 1, _PCH),
        ny.reshape(_B, _NJ, 1, _PCH),
        nz.reshape(_B, _NJ, 1, _PCH),
    )

    idxflat = idxg.transpose(0, 2, 1, 3).reshape(_ROWS)
    cnt = cntg.reshape(_B, _P, 1)

    table = jnp.concatenate(
        [xyz, features.transpose(0, 2, 1),
         jnp.zeros((_B, _N, 32 - 3 - _C), jnp.float32)], axis=-1
    ).reshape(_B * _N, 32)

    g = _gather(table, idxflat).reshape(_B, _S * _P, 32)

    w1p = jnp.zeros((32, 32), jnp.float32).at[:3 + _C, :].set(W1.T)
    feats = _mlp_pool(
        g, nx[:, None, :], ny[:, None, :], nz[:, None, :], cnt,
        w1p, b1[None, :], W2.T, b2[None, :], W3.T, b3[None, :])

    new_xyz = jnp.stack([nx, ny, nz], axis=-1)
    new_features = feats.transpose(0, 2, 1)
    return (new_xyz, new_features)


# speedup vs baseline: 17.8484x; 1.0803x over previous
import functools

import numpy as np
import jax
import jax.numpy as jnp
from jax import lax
from jax.experimental import pallas as pl
from jax.experimental.pallas import tpu as pltpu
from jax.experimental.pallas import tpu_sc as plsc

_B, _N, _C = 4, 8192, 16
_P, _S = 1024, 32
_R2 = np.float32(0.8 * 0.8)
_PCH = 128
_NJ = _P // _PCH


def _fps_body(x_ref, y_ref, z_ref, nx_ref, ny_ref, nz_ref):
    x = x_ref[...]
    y = y_ref[...]
    z = z_ref[...]
    lane = lax.broadcasted_iota(jnp.int32, (_B, _N), 1)
    lx = x[:, 0:1]
    ly = y[:, 0:1]
    lz = z[:, 0:1]
    pcol = lax.broadcasted_iota(jnp.int32, (_B, _P), 1)
    nx0 = jnp.broadcast_to(lx, (_B, _P))
    ny0 = jnp.broadcast_to(ly, (_B, _P))
    nz0 = jnp.broadcast_to(lz, (_B, _P))
    dists0 = jnp.full((_B, _N), 1e10, dtype=jnp.float32)

    def body(i, carry):
        dists, lx, ly, lz, nxa, nya, nza = carry
        dx = x - lx
        dy = y - ly
        dz = z - lz
        d = dx * dx + dy * dy + dz * dz
        dists = jnp.minimum(dists, d)
        nxt = jnp.argmax(dists, axis=-1)[:, None]
        sel = lane == nxt
        lx = jnp.sum(jnp.where(sel, x, 0.0), axis=-1, keepdims=True)
        ly = jnp.sum(jnp.where(sel, y, 0.0), axis=-1, keepdims=True)
        lz = jnp.sum(jnp.where(sel, z, 0.0), axis=-1, keepdims=True)
        slot = pcol == i
        nxa = jnp.where(slot, lx, nxa)
        nya = jnp.where(slot, ly, nya)
        nza = jnp.where(slot, lz, nza)
        return dists, lx, ly, lz, nxa, nya, nza

    carry = lax.fori_loop(1, _P, body, (dists0, lx, ly, lz, nx0, ny0, nz0))
    nx_ref[...] = carry[4]
    ny_ref[...] = carry[5]
    nz_ref[...] = carry[6]


def _fps(xs, ys, zs):
    out = jax.ShapeDtypeStruct((_B, _P), jnp.float32)
    return pl.pallas_call(
        _fps_body,
        out_shape=(out, out, out),
    )(xs, ys, zs)


def _bq_body(xc_ref, yc_ref, zc_ref, cx_ref, cy_ref, cz_ref, idx_ref, cnt_ref):
    b = pl.program_id(0)
    xc = xc_ref[0]
    yc = yc_ref[0]
    zc = zc_ref[0]
    cx = cx_ref[0, 0]
    cy = cy_ref[0, 0]
    cz = cz_ref[0, 0]
    dx = xc - cx
    dy = yc - cy
    dz = zc - cz
    d2 = dx * dx + dy * dy + dz * dz
    within = d2 < _R2
    pid = lax.broadcasted_iota(jnp.int32, (_N, _PCH), 0)
    keys = jnp.where(within, pid, _N)
    cnt_ref[0, 0] = jnp.sum(within.astype(jnp.int32), axis=0, keepdims=True)
    cols = []
    for _ in range(_S):
        m = jnp.min(keys, axis=0, keepdims=True)
        cols.append(m)
        keys = jnp.where(keys == m, _N, keys)
    idx = jnp.concatenate(cols, axis=0)
    first = jnp.where(cols[0] == _N, 0, cols[0])
    idx = jnp.where(idx == _N, first, idx)
    idx_ref[0, 0] = idx + b * _N


def _ballquery(xc, yc, zc, cx, cy, cz):
    pt_spec = pl.BlockSpec((1, _N, 1), lambda b, j: (b, 0, 0))
    cen_spec = pl.BlockSpec((1, 1, 1, _PCH), lambda b, j: (b, j, 0, 0))
    return pl.pallas_call(
        _bq_body,
        grid=(_B, _NJ),
        in_specs=[pt_spec, pt_spec, pt_spec, cen_spec, cen_spec, cen_spec],
        out_specs=[
            pl.BlockSpec((1, 1, _S, _PCH), lambda b, j: (b, j, 0, 0)),
            pl.BlockSpec((1, 1, 1, _PCH), lambda b, j: (b, j, 0, 0)),
        ],
        out_shape=[
            jax.ShapeDtypeStruct((_B, _NJ, _S, _PCH), jnp.int32),
            jax.ShapeDtypeStruct((_B, _NJ, 1, _PCH), jnp.int32),
        ],
    )(xc, yc, zc, cx, cy, cz)


_NW = 32
_ROWS = _B * _S * _P
_RPW = _ROWS // _NW
_GCH = 128
_HALF = 2048


def _gather(table, idxflat):
    mesh = plsc.VectorSubcoreMesh(core_axis_name="c", subcore_axis_name="s")

    @functools.partial(
        pl.kernel,
        out_type=jax.ShapeDtypeStruct((_ROWS, 32), jnp.float32),
        mesh=mesh,
        compiler_params=pltpu.CompilerParams(use_tc_tiling_on_sc=False),
        scratch_types=[
            pltpu.VMEM((_RPW,), jnp.int32),
            pltpu.VMEM((_HALF, 32), jnp.float32),
            pltpu.SemaphoreType.DMA,
        ],
    )
    def k(table_hbm, idx_hbm, out_hbm, idx_v, rows_v, sem):
        wid = lax.axis_index("s") * 2 + lax.axis_index("c")
        base = wid * _RPW
        pltpu.sync_copy(idx_hbm.at[pl.ds(base, _RPW)], idx_v)
        for h in range(_RPW // _HALF):
            copies = []
            for c in range(_HALF // _GCH):
                off = h * _HALF + c * _GCH
                copies.append(pltpu.async_copy(
                    table_hbm.at[idx_v.at[pl.ds(off, _GCH)]],
                    rows_v.at[pl.ds(c * _GCH, _GCH)], sem))
            for cp in copies:
                cp.wait()
            pltpu.sync_copy(rows_v, out_hbm.at[pl.ds(base + h * _HALF, _HALF)])

    return k(table, idxflat)


def _mlp_body(g_ref, nx_ref, ny_ref, nz_ref, cnt_ref,
              w1_ref, b1_ref, w2_ref, b2_ref, w3_ref, b3_ref, out_ref):
    g = g_ref[0]
    nx = nx_ref[0]
    ny = ny_ref[0]
    nz = nz_ref[0]
    cen = jnp.concatenate(
        [nx[:, :, None], ny[:, :, None], nz[:, :, None],
         jnp.zeros((1, _P, 29), jnp.float32)], axis=2)
    x = (g.reshape(_S, _P, 32) - cen).reshape(_S * _P, 32)
    h = jnp.maximum(
        jnp.dot(x, w1_ref[...], preferred_element_type=jnp.float32)
        + b1_ref[...], 0.0)
    h = jnp.maximum(
        jnp.dot(h, w2_ref[...], preferred_element_type=jnp.float32)
        + b2_ref[...], 0.0)
    h = jnp.maximum(
        jnp.dot(h, w3_ref[...], preferred_element_type=jnp.float32)
        + b3_ref[...], 0.0)
    red = jnp.max(h.reshape(_S, _P, 64), axis=0)
    mask = cnt_ref[0] > 0
    out_ref[0] = jnp.where(mask, red, 0.0)


def _mlp_pool(g, nx, ny, nz, cnt, w1p, b1, w2t, b2, w3t, b3):
    full = lambda shape: pl.BlockSpec(shape, lambda b: (0,) * len(shape))
    return pl.pallas_call(
        _mlp_body,
        grid=(_B,),
        in_specs=[
            pl.BlockSpec((1, _S * _P, 32), lambda b: (b, 0, 0)),
            pl.BlockSpec((1, 1, _P), lambda b: (b, 0, 0)),
            pl.BlockSpec((1, 1, _P), lambda b: (b, 0, 0)),
            pl.BlockSpec((1, 1, _P), lambda b: (b, 0, 0)),
            pl.BlockSpec((1, _P, 1), lambda b: (b, 0, 0)),
            full((32, 32)), full((1, 32)),
            full((32, 32)), full((1, 32)),
            full((32, 64)), full((1, 64)),
        ],
        out_specs=pl.BlockSpec((1, _P, 64), lambda b: (b, 0, 0)),
        out_shape=jax.ShapeDtypeStruct((_B, _P, 64), jnp.float32),
    )(g, nx, ny, nz, cnt, w1p, b1, w2t, b2, w3t, b3)


def kernel(xyz, features, W1, b1, W2, b2, W3, b3):
    xs = xyz[:, :, 0]
    ys = xyz[:, :, 1]
    zs = xyz[:, :, 2]

    nx, ny, nz = _fps(xs, ys, zs)

    idxg, cntg = _ballquery(
        xs[:, :, None], ys[:, :, None], zs[:, :, None],
        nx.reshape(_B, _NJ, 1, _PCH),
        ny.reshape(_B, _NJ, 1, _PCH),
        nz.reshape(_B, _NJ, 1, _PCH),
    )

    idxflat = idxg.transpose(0, 2, 1, 3).reshape(_ROWS)
    cnt = cntg.reshape(_B, _P, 1)

    table = jnp.concatenate(
        [xyz, features.transpose(0, 2, 1),
         jnp.zeros((_B, _N, 32 - 3 - _C), jnp.float32)], axis=-1
    ).reshape(_B * _N, 32)

    g = _gather(table, idxflat).reshape(_B, _S * _P, 32)

    w1p = jnp.zeros((32, 32), jnp.float32).at[:3 + _C, :].set(W1.T)
    feats = _mlp_pool(
        g, nx[:, None, :], ny[:, None, :], nz[:, None, :], cnt,
        w1p, b1[None, :], W2.T, b2[None, :], W3.T, b3[None, :])

    new_xyz = jnp.stack([nx, ny, nz], axis=-1)
    new_features = feats.transpose(0, 2, 1)
    return (new_xyz, new_features)


# speedup vs baseline: 26.3941x; 1.4788x over previous
import functools

import numpy as np
import jax
import jax.numpy as jnp
from jax import lax
from jax.experimental import pallas as pl
from jax.experimental.pallas import tpu as pltpu
from jax.experimental.pallas import tpu_sc as plsc

_B, _N, _C = 4, 8192, 16
_P, _S = 1024, 32
_R2 = np.float32(0.8 * 0.8)
_PCH = 128
_NJ = _P // _PCH
_NW = 32


def _fps_body(x_ref, y_ref, z_ref, nx_ref, ny_ref, nz_ref):
    x = x_ref[...]
    y = y_ref[...]
    z = z_ref[...]
    lane = lax.broadcasted_iota(jnp.int32, (_B, _N), 1)
    lx = x[:, 0:1]
    ly = y[:, 0:1]
    lz = z[:, 0:1]
    pcol = lax.broadcasted_iota(jnp.int32, (_B, _P), 1)
    nx0 = jnp.broadcast_to(lx, (_B, _P))
    ny0 = jnp.broadcast_to(ly, (_B, _P))
    nz0 = jnp.broadcast_to(lz, (_B, _P))
    dists0 = jnp.full((_B, _N), 1e10, dtype=jnp.float32)

    def body(i, carry):
        dists, lx, ly, lz, nxa, nya, nza = carry
        dx = x - lx
        dy = y - ly
        dz = z - lz
        d = dx * dx + dy * dy + dz * dz
        dists = jnp.minimum(dists, d)
        nxt = jnp.argmax(dists, axis=-1)[:, None]
        sel = lane == nxt
        lx = jnp.sum(jnp.where(sel, x, 0.0), axis=-1, keepdims=True)
        ly = jnp.sum(jnp.where(sel, y, 0.0), axis=-1, keepdims=True)
        lz = jnp.sum(jnp.where(sel, z, 0.0), axis=-1, keepdims=True)
        slot = pcol == i
        nxa = jnp.where(slot, lx, nxa)
        nya = jnp.where(slot, ly, nya)
        nza = jnp.where(slot, lz, nza)
        return dists, lx, ly, lz, nxa, nya, nza

    carry = lax.fori_loop(1, _P, body, (dists0, lx, ly, lz, nx0, ny0, nz0))
    nx_ref[...] = carry[4]
    ny_ref[...] = carry[5]
    nz_ref[...] = carry[6]


def _fps(xs, ys, zs):
    out = jax.ShapeDtypeStruct((_B, _P), jnp.float32)
    return pl.pallas_call(
        _fps_body,
        out_shape=(out, out, out),
    )(xs, ys, zs)


_NW32 = _N // 32


def _bq_body(xc_ref, yc_ref, zc_ref, cx_ref, cy_ref, cz_ref, bits_ref, cnt_ref):
    xc = xc_ref[0]
    yc = yc_ref[0]
    zc = zc_ref[0]
    cx = cx_ref[0, 0]
    cy = cy_ref[0, 0]
    cz = cz_ref[0, 0]
    dx = xc - cx
    dy = yc - cy
    dz = zc - cz
    d2 = dx * dx + dy * dy + dz * dz
    within = d2 < _R2
    wi = within.astype(jnp.int32)
    cnt_ref[0, 0] = jnp.sum(wi, axis=0, keepdims=True)
    sub = lax.broadcasted_iota(jnp.int32, (_N, _PCH), 0)
    sh = wi << (sub & 31)
    bits_ref[0, 0] = jnp.sum(sh.reshape(_NW32, 32, _PCH), axis=1)


def _ballquery(xc, yc, zc, cx, cy, cz):
    pt_spec = pl.BlockSpec((1, _N, 1), lambda b, j: (b, 0, 0))
    cen_spec = pl.BlockSpec((1, 1, 1, _PCH), lambda b, j: (b, j, 0, 0))
    return pl.pallas_call(
        _bq_body,
        grid=(_B, _NJ),
        in_specs=[pt_spec, pt_spec, pt_spec, cen_spec, cen_spec, cen_spec],
        out_specs=[
            pl.BlockSpec((1, 1, _NW32, _PCH), lambda b, j: (b, j, 0, 0)),
            pl.BlockSpec((1, 1, 1, _PCH), lambda b, j: (b, j, 0, 0)),
        ],
        out_shape=[
            jax.ShapeDtypeStruct((_B, _NJ, _NW32, _PCH), jnp.int32),
            jax.ShapeDtypeStruct((_B, _NJ, 1, _PCH), jnp.int32),
        ],
    )(xc, yc, zc, cx, cy, cz)


_DBJ = np.array([0, 1, 28, 2, 29, 14, 24, 3, 30, 22, 20, 15, 25, 17, 4, 8,
                 31, 27, 13, 23, 21, 19, 16, 7, 26, 12, 18, 6, 11, 5, 10, 9],
                dtype=np.int32)
_CPW = (_B * _P) // _NW
_NCH = _NW32 // 16


def _extract_sc(bits_rows, dbj):
    mesh = plsc.VectorSubcoreMesh(core_axis_name="c", subcore_axis_name="s")

    @functools.partial(
        pl.kernel,
        out_type=jax.ShapeDtypeStruct((_B * _P, _S), jnp.int32),
        mesh=mesh,
        compiler_params=pltpu.CompilerParams(use_tc_tiling_on_sc=False, needs_layout_passes=False),
        scratch_types=[
            pltpu.VMEM((_CPW * _NCH, 16), jnp.int32),
            pltpu.VMEM((_CPW, _S), jnp.int32),
            pltpu.VMEM((32,), jnp.int32),
            pltpu.VMEM((16,), jnp.int32),
        ],
    )
    def k(bits_hbm, dbj_hbm, out_hbm, bits_v, out_v, tbl_v, vtmp_v):
        wid = lax.axis_index("s") * 2 + lax.axis_index("c")
        base = wid * _CPW
        boff = (base // _P) * _N
        pltpu.sync_copy(dbj_hbm, tbl_v)
        pltpu.sync_copy(bits_hbm.at[pl.ds(base * _NCH, _CPW * _NCH)], bits_v)
        lanes = lax.iota(jnp.int32, 16)
        zero16 = jnp.zeros((16,), jnp.int32)
        big = jnp.int32(2 ** 30)

        def ctz16(t):
            i = ((t * 0x077CB531) >> 27) & 31
            return plsc.load_gather(tbl_v, [i])

        def col_body(col, carry):
            colv = zero16 + col

            def chunk_body(ch, st):
                count, first = st
                v = bits_v[col * _NCH + ch]
                p1 = v - ((v >> 1) & 0x55555555)
                p2 = (p1 & 0x33333333) + ((p1 >> 2) & 0x33333333)
                p3 = (p2 + (p2 >> 4)) & 0x0F0F0F0F
                wp = ((p3 * 0x01010101) >> 24) & 63
                baser = count + plsc.cumsum(wp) - wp
                posbase = ch * 512 + lanes * 32 + boff
                fc = jnp.min(jnp.where(v != 0, ctz16(v & (-v)) + posbase, big))
                first = lax.select((count == 0) & (fc < big), fc, first)

                need = jnp.maximum(jnp.int32(_S) - baser, 0)
                iters = jnp.max(jnp.minimum(wp, need))
                vtmp_v[...] = v

                def in_body(kk, uu):
                    v2 = vtmp_v[...]
                    t = v2 & (-v2)
                    pos = ctz16(t) + posbase
                    slot = baser + kk
                    valid = (t != 0) & (slot < _S)
                    plsc.store_scatter(out_v, [colv, slot], pos, mask=valid)
                    vtmp_v[...] = v2 ^ t
                    return uu

                lax.fori_loop(0, iters, in_body, 0)
                return (count + jnp.sum(wp), first)

            count, first = lax.fori_loop(
                0, _NCH, chunk_body, (jnp.int32(0), boff))
            fillv = zero16 + first
            for g in range(2):
                sl = lanes + g * 16
                plsc.store_scatter(out_v, [colv, sl], fillv, mask=sl >= count)
            return carry

        lax.fori_loop(0, _CPW, col_body, 0)
        pltpu.sync_copy(out_v, out_hbm.at[pl.ds(base, _CPW)])

    return k(bits_rows, dbj)


_ROWS = _B * _S * _P
_RPW = _ROWS // _NW
_GCH = 128
_HALF = 2048


def _gather(table, idxflat):
    mesh = plsc.VectorSubcoreMesh(core_axis_name="c", subcore_axis_name="s")

    @functools.partial(
        pl.kernel,
        out_type=jax.ShapeDtypeStruct((_ROWS, 32), jnp.float32),
        mesh=mesh,
        compiler_params=pltpu.CompilerParams(use_tc_tiling_on_sc=False, needs_layout_passes=False),
        scratch_types=[
            pltpu.VMEM((_RPW,), jnp.int32),
            pltpu.VMEM((_HALF, 32), jnp.float32),
            pltpu.SemaphoreType.DMA,
        ],
    )
    def k(table_hbm, idx_hbm, out_hbm, idx_v, rows_v, sem):
        wid = lax.axis_index("s") * 2 + lax.axis_index("c")
        base = wid * _RPW
        pltpu.sync_copy(idx_hbm.at[pl.ds(base, _RPW)], idx_v)
        for h in range(_RPW // _HALF):
            copies = []
            for c in range(_HALF // _GCH):
                off = h * _HALF + c * _GCH
                copies.append(pltpu.async_copy(
                    table_hbm.at[idx_v.at[pl.ds(off, _GCH)]],
                    rows_v.at[pl.ds(c * _GCH, _GCH)], sem))
            for cp in copies:
                cp.wait()
            pltpu.sync_copy(rows_v, out_hbm.at[pl.ds(base + h * _HALF, _HALF)])

    return k(table, idxflat)


def _mlp_body(g_ref, nx_ref, ny_ref, nz_ref, cnt_ref,
              w1_ref, b1_ref, w2_ref, b2_ref, w3_ref, b3_ref, out_ref):
    g = g_ref[0]
    nx = nx_ref[0]
    ny = ny_ref[0]
    nz = nz_ref[0]
    cen = jnp.concatenate(
        [nx, ny, nz, jnp.zeros((_P, 29), jnp.float32)], axis=1)
    x = (g.reshape(_P, _S, 32) - cen[:, None, :]).reshape(_S * _P, 32)
    h = jnp.maximum(
        jnp.dot(x, w1_ref[...], preferred_element_type=jnp.float32)
        + b1_ref[...], 0.0)
    h = jnp.maximum(
        jnp.dot(h, w2_ref[...], preferred_element_type=jnp.float32)
        + b2_ref[...], 0.0)
    h = jnp.maximum(
        jnp.dot(h, w3_ref[...], preferred_element_type=jnp.float32)
        + b3_ref[...], 0.0)
    red = jnp.max(h.reshape(_P, _S, 64), axis=1)
    mask = cnt_ref[0] > 0
    out_ref[0] = jnp.where(mask, red, 0.0)


def _mlp_pool(g, nx, ny, nz, cnt, w1p, b1, w2t, b2, w3t, b3):
    full = lambda shape: pl.BlockSpec(shape, lambda b: (0,) * len(shape))
    return pl.pallas_call(
        _mlp_body,
        grid=(_B,),
        in_specs=[
            pl.BlockSpec((1, _S * _P, 32), lambda b: (b, 0, 0)),
            pl.BlockSpec((1, _P, 1), lambda b: (b, 0, 0)),
            pl.BlockSpec((1, _P, 1), lambda b: (b, 0, 0)),
            pl.BlockSpec((1, _P, 1), lambda b: (b, 0, 0)),
            pl.BlockSpec((1, _P, 1), lambda b: (b, 0, 0)),
            full((32, 32)), full((1, 32)),
            full((32, 32)), full((1, 32)),
            full((32, 64)), full((1, 64)),
        ],
        out_specs=pl.BlockSpec((1, _P, 64), lambda b: (b, 0, 0)),
        out_shape=jax.ShapeDtypeStruct((_B, _P, 64), jnp.float32),
    )(g, nx, ny, nz, cnt, w1p, b1, w2t, b2, w3t, b3)


def kernel(xyz, features, W1, b1, W2, b2, W3, b3):
    xs = xyz[:, :, 0]
    ys = xyz[:, :, 1]
    zs = xyz[:, :, 2]

    nx, ny, nz = _fps(xs, ys, zs)

    bits, cntg = _ballquery(
        xs[:, :, None], ys[:, :, None], zs[:, :, None],
        nx.reshape(_B, _NJ, 1, _PCH),
        ny.reshape(_B, _NJ, 1, _PCH),
        nz.reshape(_B, _NJ, 1, _PCH),
    )

    bits_rows = bits.transpose(0, 1, 3, 2).reshape(_B * _P * _NCH, 16)
    idxflat = _extract_sc(bits_rows, jnp.asarray(_DBJ)).reshape(_ROWS)
    cnt = cntg.reshape(_B, _P, 1)

    table = jnp.concatenate(
        [xyz, features.transpose(0, 2, 1),
         jnp.zeros((_B, _N, 32 - 3 - _C), jnp.float32)], axis=-1
    ).reshape(_B * _N, 32)

    g = _gather(table, idxflat).reshape(_B, _S * _P, 32)

    w1p = jnp.zeros((32, 32), jnp.float32).at[:3 + _C, :].set(W1.T)
    feats = _mlp_pool(
        g, nx[:, :, None], ny[:, :, None], nz[:, :, None], cnt,
        w1p, b1[None, :], W2.T, b2[None, :], W3.T, b3[None, :])

    new_xyz = jnp.stack([nx, ny, nz], axis=-1)
    new_features = feats.transpose(0, 2, 1)
    return (new_xyz, new_features)


# speedup vs baseline: 27.1076x; 1.0270x over previous
import functools

import numpy as np
import jax
import jax.numpy as jnp
from jax import lax
from jax.experimental import pallas as pl
from jax.experimental.pallas import tpu as pltpu
from jax.experimental.pallas import tpu_sc as plsc

_B, _N, _C = 4, 8192, 16
_P, _S = 1024, 32
_R2 = np.float32(0.8 * 0.8)
_PCH = 128
_NJ = _P // _PCH
_NW = 32


_KF = 8
_NL = _N // _KF


def _fps_body(x_ref, y_ref, z_ref, nx_ref, ny_ref, nz_ref):
    x = x_ref[...]
    y = y_ref[...]
    z = z_ref[...]
    shp = (_B, _KF, _NL)
    pid = (lax.broadcasted_iota(jnp.int32, shp, 1) * _NL
           + lax.broadcasted_iota(jnp.int32, shp, 2))
    lx = x[:, 0:1, 0:1]
    ly = y[:, 0:1, 0:1]
    lz = z[:, 0:1, 0:1]
    pcol = lax.broadcasted_iota(jnp.int32, (_B, 1, _P), 2)
    nx0 = jnp.broadcast_to(lx, (_B, 1, _P))
    ny0 = jnp.broadcast_to(ly, (_B, 1, _P))
    nz0 = jnp.broadcast_to(lz, (_B, 1, _P))
    dists0 = jnp.full(shp, 1e10, dtype=jnp.float32)

    def body(i, carry):
        dists, lx, ly, lz, nxa, nya, nza = carry
        dx = x - lx
        dy = y - ly
        dz = z - lz
        d = dx * dx + dy * dy + dz * dz
        dists = jnp.minimum(dists, d)
        m = jnp.max(dists, axis=(1, 2), keepdims=True)
        cand = jnp.where(dists == m, pid, _N)
        nxt = jnp.min(cand, axis=(1, 2), keepdims=True)
        sel = pid == nxt
        lx = jnp.sum(jnp.where(sel, x, 0.0), axis=(1, 2), keepdims=True)
        ly = jnp.sum(jnp.where(sel, y, 0.0), axis=(1, 2), keepdims=True)
        lz = jnp.sum(jnp.where(sel, z, 0.0), axis=(1, 2), keepdims=True)
        slot = pcol == i
        nxa = jnp.where(slot, lx, nxa)
        nya = jnp.where(slot, ly, nya)
        nza = jnp.where(slot, lz, nza)
        return dists, lx, ly, lz, nxa, nya, nza

    carry = lax.fori_loop(1, _P, body,
                          (dists0, lx, ly, lz, nx0, ny0, nz0))
    nx_ref[...] = carry[4]
    ny_ref[...] = carry[5]
    nz_ref[...] = carry[6]


def _fps(xs, ys, zs):
    out = jax.ShapeDtypeStruct((_B, 1, _P), jnp.float32)
    r = pl.pallas_call(
        _fps_body,
        out_shape=(out, out, out),
    )(xs.reshape(_B, _KF, _NL), ys.reshape(_B, _KF, _NL),
      zs.reshape(_B, _KF, _NL))
    return tuple(t.reshape(_B, _P) for t in r)


_NW32 = _N // 32


def _bq_body(xc_ref, yc_ref, zc_ref, cx_ref, cy_ref, cz_ref, bits_ref, cnt_ref):
    xc = xc_ref[0]
    yc = yc_ref[0]
    zc = zc_ref[0]
    cx = cx_ref[0, 0]
    cy = cy_ref[0, 0]
    cz = cz_ref[0, 0]
    dx = xc - cx
    dy = yc - cy
    dz = zc - cz
    d2 = dx * dx + dy * dy + dz * dz
    within = d2 < _R2
    wi = within.astype(jnp.int32)
    cnt_ref[0, 0] = jnp.sum(wi, axis=0, keepdims=True)
    sub = lax.broadcasted_iota(jnp.int32, (_N, _PCH), 0)
    sh = wi << (sub & 31)
    bits_ref[0, 0] = jnp.sum(sh.reshape(_NW32, 32, _PCH), axis=1)


def _ballquery(xc, yc, zc, cx, cy, cz):
    pt_spec = pl.BlockSpec((1, _N, 1), lambda b, j: (b, 0, 0))
    cen_spec = pl.BlockSpec((1, 1, 1, _PCH), lambda b, j: (b, j, 0, 0))
    return pl.pallas_call(
        _bq_body,
        grid=(_B, _NJ),
        in_specs=[pt_spec, pt_spec, pt_spec, cen_spec, cen_spec, cen_spec],
        out_specs=[
            pl.BlockSpec((1, 1, _NW32, _PCH), lambda b, j: (b, j, 0, 0)),
            pl.BlockSpec((1, 1, 1, _PCH), lambda b, j: (b, j, 0, 0)),
        ],
        out_shape=[
            jax.ShapeDtypeStruct((_B, _NJ, _NW32, _PCH), jnp.int32),
            jax.ShapeDtypeStruct((_B, _NJ, 1, _PCH), jnp.int32),
        ],
    )(xc, yc, zc, cx, cy, cz)


_DBJ = np.array([0, 1, 28, 2, 29, 14, 24, 3, 30, 22, 20, 15, 25, 17, 4, 8,
                 31, 27, 13, 23, 21, 19, 16, 7, 26, 12, 18, 6, 11, 5, 10, 9],
                dtype=np.int32)
_CPW = (_B * _P) // _NW
_NCH = _NW32 // 16


def _extract_sc(bits_rows, dbj):
    mesh = plsc.VectorSubcoreMesh(core_axis_name="c", subcore_axis_name="s")

    @functools.partial(
        pl.kernel,
        out_type=jax.ShapeDtypeStruct((_B * _P, _S), jnp.int32),
        mesh=mesh,
        compiler_params=pltpu.CompilerParams(use_tc_tiling_on_sc=False, needs_layout_passes=False),
        scratch_types=[
            pltpu.VMEM((_CPW * _NCH, 16), jnp.int32),
            pltpu.VMEM((_CPW, _S), jnp.int32),
            pltpu.VMEM((32,), jnp.int32),
            pltpu.VMEM((16,), jnp.int32),
        ],
    )
    def k(bits_hbm, dbj_hbm, out_hbm, bits_v, out_v, tbl_v, vtmp_v):
        wid = lax.axis_index("s") * 2 + lax.axis_index("c")
        base = wid * _CPW
        boff = (base // _P) * _N
        pltpu.sync_copy(dbj_hbm, tbl_v)
        pltpu.sync_copy(bits_hbm.at[pl.ds(base * _NCH, _CPW * _NCH)], bits_v)
        lanes = lax.iota(jnp.int32, 16)
        zero16 = jnp.zeros((16,), jnp.int32)
        big = jnp.int32(2 ** 30)

        def ctz16(t):
            i = ((t * 0x077CB531) >> 27) & 31
            return plsc.load_gather(tbl_v, [i])

        def col_body(col, carry):
            colv = zero16 + col

            def chunk_body(ch, st):
                count, first = st
                v = bits_v[col * _NCH + ch]
                p1 = v - ((v >> 1) & 0x55555555)
                p2 = (p1 & 0x33333333) + ((p1 >> 2) & 0x33333333)
                p3 = (p2 + (p2 >> 4)) & 0x0F0F0F0F
                wp = ((p3 * 0x01010101) >> 24) & 63
                baser = count + plsc.cumsum(wp) - wp
                posbase = ch * 512 + lanes * 32 + boff
                fc = jnp.min(jnp.where(v != 0, ctz16(v & (-v)) + posbase, big))
                first = lax.select((count == 0) & (fc < big), fc, first)

                need = jnp.maximum(jnp.int32(_S) - baser, 0)
                iters = jnp.max(jnp.minimum(wp, need))
                vtmp_v[...] = v

                def in_body(kk, uu):
                    v2 = vtmp_v[...]
                    t = v2 & (-v2)
                    pos = ctz16(t) + posbase
                    slot = baser + kk
                    valid = (t != 0) & (slot < _S)
                    plsc.store_scatter(out_v, [colv, slot], pos, mask=valid)
                    vtmp_v[...] = v2 ^ t
                    return uu

                lax.fori_loop(0, iters, in_body, 0)
                return (count + jnp.sum(wp), first)

            count, first = lax.fori_loop(
                0, _NCH, chunk_body, (jnp.int32(0), boff))
            fillv = zero16 + first
            for g in range(2):
                sl = lanes + g * 16
                plsc.store_scatter(out_v, [colv, sl], fillv, mask=sl >= count)
            return carry

        lax.fori_loop(0, _CPW, col_body, 0)
        pltpu.sync_copy(out_v, out_hbm.at[pl.ds(base, _CPW)])

    return k(bits_rows, dbj)


_ROWS = _B * _S * _P
_RPW = _ROWS // _NW
_GCH = 128
_HALF = 2048


def _gather(table, idxflat):
    mesh = plsc.VectorSubcoreMesh(core_axis_name="c", subcore_axis_name="s")

    @functools.partial(
        pl.kernel,
        out_type=jax.ShapeDtypeStruct((_ROWS, 32), jnp.float32),
        mesh=mesh,
        compiler_params=pltpu.CompilerParams(use_tc_tiling_on_sc=False, needs_layout_passes=False),
        scratch_types=[
            pltpu.VMEM((_RPW,), jnp.int32),
            pltpu.VMEM((_HALF, 32), jnp.float32),
            pltpu.SemaphoreType.DMA,
        ],
    )
    def k(table_hbm, idx_hbm, out_hbm, idx_v, rows_v, sem):
        wid = lax.axis_index("s") * 2 + lax.axis_index("c")
        base = wid * _RPW
        pltpu.sync_copy(idx_hbm.at[pl.ds(base, _RPW)], idx_v)
        for h in range(_RPW // _HALF):
            copies = []
            for c in range(_HALF // _GCH):
                off = h * _HALF + c * _GCH
                copies.append(pltpu.async_copy(
                    table_hbm.at[idx_v.at[pl.ds(off, _GCH)]],
                    rows_v.at[pl.ds(c * _GCH, _GCH)], sem))
            for cp in copies:
                cp.wait()
            pltpu.sync_copy(rows_v, out_hbm.at[pl.ds(base + h * _HALF, _HALF)])

    return k(table, idxflat)


def _mlp_body(g_ref, nx_ref, ny_ref, nz_ref, cnt_ref,
              w1_ref, b1_ref, w2_ref, b2_ref, w3_ref, b3_ref, out_ref):
    g = g_ref[0]
    nx = nx_ref[0]
    ny = ny_ref[0]
    nz = nz_ref[0]
    cen = jnp.concatenate(
        [nx, ny, nz, jnp.zeros((_P, 29), jnp.float32)], axis=1)
    x = (g.reshape(_P, _S, 32) - cen[:, None, :]).reshape(_S * _P, 32)
    h = jnp.maximum(
        jnp.dot(x, w1_ref[...], preferred_element_type=jnp.float32)
        + b1_ref[...], 0.0)
    h = jnp.maximum(
        jnp.dot(h, w2_ref[...], preferred_element_type=jnp.float32)
        + b2_ref[...], 0.0)
    h = jnp.maximum(
        jnp.dot(h, w3_ref[...], preferred_element_type=jnp.float32)
        + b3_ref[...], 0.0)
    red = jnp.max(h.reshape(_P, _S, 64), axis=1)
    mask = cnt_ref[0] > 0
    out_ref[0] = jnp.where(mask, red, 0.0)


def _mlp_pool(g, nx, ny, nz, cnt, w1p, b1, w2t, b2, w3t, b3):
    full = lambda shape: pl.BlockSpec(shape, lambda b: (0,) * len(shape))
    return pl.pallas_call(
        _mlp_body,
        grid=(_B,),
        in_specs=[
            pl.BlockSpec((1, _S * _P, 32), lambda b: (b, 0, 0)),
            pl.BlockSpec((1, _P, 1), lambda b: (b, 0, 0)),
            pl.BlockSpec((1, _P, 1), lambda b: (b, 0, 0)),
            pl.BlockSpec((1, _P, 1), lambda b: (b, 0, 0)),
            pl.BlockSpec((1, _P, 1), lambda b: (b, 0, 0)),
            full((32, 32)), full((1, 32)),
            full((32, 32)), full((1, 32)),
            full((32, 64)), full((1, 64)),
        ],
        out_specs=pl.BlockSpec((1, _P, 64), lambda b: (b, 0, 0)),
        out_shape=jax.ShapeDtypeStruct((_B, _P, 64), jnp.float32),
    )(g, nx, ny, nz, cnt, w1p, b1, w2t, b2, w3t, b3)


def kernel(xyz, features, W1, b1, W2, b2, W3, b3):
    xs = xyz[:, :, 0]
    ys = xyz[:, :, 1]
    zs = xyz[:, :, 2]

    nx, ny, nz = _fps(xs, ys, zs)

    bits, cntg = _ballquery(
        xs[:, :, None], ys[:, :, None], zs[:, :, None],
        nx.reshape(_B, _NJ, 1, _PCH),
        ny.reshape(_B, _NJ, 1, _PCH),
        nz.reshape(_B, _NJ, 1, _PCH),
    )

    bits_rows = bits.transpose(0, 1, 3, 2).reshape(_B * _P * _NCH, 16)
    idxflat = _extract_sc(bits_rows, jnp.asarray(_DBJ)).reshape(_ROWS)
    cnt = cntg.reshape(_B, _P, 1)

    table = jnp.concatenate(
        [xyz, features.transpose(0, 2, 1),
         jnp.zeros((_B, _N, 32 - 3 - _C), jnp.float32)], axis=-1
    ).reshape(_B * _N, 32)

    g = _gather(table, idxflat).reshape(_B, _S * _P, 32)

    w1p = jnp.zeros((32, 32), jnp.float32).at[:3 + _C, :].set(W1.T)
    feats = _mlp_pool(
        g, nx[:, :, None], ny[:, :, None], nz[:, :, None], cnt,
        w1p, b1[None, :], W2.T, b2[None, :], W3.T, b3[None, :])

    new_xyz = jnp.stack([nx, ny, nz], axis=-1)
    new_features = feats.transpose(0, 2, 1)
    return (new_xyz, new_features)


# speedup vs baseline: 27.7123x; 1.0223x over previous
import functools

import numpy as np
import jax
import jax.numpy as jnp
from jax import lax
from jax.experimental import pallas as pl
from jax.experimental.pallas import tpu as pltpu
from jax.experimental.pallas import tpu_sc as plsc

_B, _N, _C = 4, 8192, 16
_P, _S = 1024, 32
_R2 = np.float32(0.8 * 0.8)
_PCH = 128
_NJ = _P // _PCH
_NW = 32


_KF = 8
_NL = _N // _KF


def _fps_body(x_ref, y_ref, z_ref, nx_ref, ny_ref, nz_ref):
    x = x_ref[...]
    y = y_ref[...]
    z = z_ref[...]
    shp = (_B, _KF, _NL)
    pid = (lax.broadcasted_iota(jnp.int32, shp, 1) * _NL
           + lax.broadcasted_iota(jnp.int32, shp, 2))
    lx = x[:, 0:1, 0:1]
    ly = y[:, 0:1, 0:1]
    lz = z[:, 0:1, 0:1]
    pcol = lax.broadcasted_iota(jnp.int32, (_B, 1, _P), 2)
    nx0 = jnp.broadcast_to(lx, (_B, 1, _P))
    ny0 = jnp.broadcast_to(ly, (_B, 1, _P))
    nz0 = jnp.broadcast_to(lz, (_B, 1, _P))
    dists0 = jnp.full(shp, 1e10, dtype=jnp.float32)

    def body(i, carry):
        dists, lx, ly, lz, nxa, nya, nza = carry
        dx = x - lx
        dy = y - ly
        dz = z - lz
        d = dx * dx + dy * dy + dz * dz
        dists = jnp.minimum(dists, d)
        m = jnp.max(dists, axis=(1, 2), keepdims=True)
        cand = jnp.where(dists == m, pid, _N)
        nxt = jnp.min(cand, axis=(1, 2), keepdims=True)
        sel = pid == nxt
        lx = jnp.sum(jnp.where(sel, x, 0.0), axis=(1, 2), keepdims=True)
        ly = jnp.sum(jnp.where(sel, y, 0.0), axis=(1, 2), keepdims=True)
        lz = jnp.sum(jnp.where(sel, z, 0.0), axis=(1, 2), keepdims=True)
        slot = pcol == i
        nxa = jnp.where(slot, lx, nxa)
        nya = jnp.where(slot, ly, nya)
        nza = jnp.where(slot, lz, nza)
        return dists, lx, ly, lz, nxa, nya, nza

    carry = lax.fori_loop(1, _P, body,
                          (dists0, lx, ly, lz, nx0, ny0, nz0))
    nx_ref[...] = carry[4]
    ny_ref[...] = carry[5]
    nz_ref[...] = carry[6]


def _fps(xs, ys, zs):
    out = jax.ShapeDtypeStruct((_B, 1, _P), jnp.float32)
    r = pl.pallas_call(
        _fps_body,
        out_shape=(out, out, out),
    )(xs.reshape(_B, _KF, _NL), ys.reshape(_B, _KF, _NL),
      zs.reshape(_B, _KF, _NL))
    return tuple(t.reshape(_B, _P) for t in r)


_NW32 = _N // 32


def _bq_body(xc_ref, yc_ref, zc_ref, cx_ref, cy_ref, cz_ref, bits_ref, cnt_ref):
    xc = xc_ref[0]
    yc = yc_ref[0]
    zc = zc_ref[0]
    cx = cx_ref[0, 0]
    cy = cy_ref[0, 0]
    cz = cz_ref[0, 0]
    dx = xc - cx
    dy = yc - cy
    dz = zc - cz
    d2 = dx * dx + dy * dy + dz * dz
    within = d2 < _R2
    wi = within.astype(jnp.int32)
    cnt_ref[0, 0] = jnp.sum(wi, axis=0, keepdims=True)
    sub = lax.broadcasted_iota(jnp.int32, (_N, _PCH), 0)
    sh = wi << (sub & 31)
    bits_ref[0, 0] = jnp.sum(sh.reshape(_NW32, 32, _PCH), axis=1)


def _ballquery(xc, yc, zc, cx, cy, cz):
    pt_spec = pl.BlockSpec((1, _N, 1), lambda b, j: (b, 0, 0))
    cen_spec = pl.BlockSpec((1, 1, 1, _PCH), lambda b, j: (b, j, 0, 0))
    return pl.pallas_call(
        _bq_body,
        grid=(_B, _NJ),
        in_specs=[pt_spec, pt_spec, pt_spec, cen_spec, cen_spec, cen_spec],
        out_specs=[
            pl.BlockSpec((1, 1, _NW32, _PCH), lambda b, j: (b, j, 0, 0)),
            pl.BlockSpec((1, 1, 1, _PCH), lambda b, j: (b, j, 0, 0)),
        ],
        out_shape=[
            jax.ShapeDtypeStruct((_B, _NJ, _NW32, _PCH), jnp.int32),
            jax.ShapeDtypeStruct((_B, _NJ, 1, _PCH), jnp.int32),
        ],
    )(xc, yc, zc, cx, cy, cz)


_DBJ = np.array([0, 1, 28, 2, 29, 14, 24, 3, 30, 22, 20, 15, 25, 17, 4, 8,
                 31, 27, 13, 23, 21, 19, 16, 7, 26, 12, 18, 6, 11, 5, 10, 9],
                dtype=np.int32)
_CPW = (_B * _P) // _NW
_NCH = _NW32 // 16


def _extract_sc(bits_rows, dbj):
    mesh = plsc.VectorSubcoreMesh(core_axis_name="c", subcore_axis_name="s")

    @functools.partial(
        pl.kernel,
        out_type=jax.ShapeDtypeStruct((_B * _P, _S), jnp.int32),
        mesh=mesh,
        compiler_params=pltpu.CompilerParams(use_tc_tiling_on_sc=False, needs_layout_passes=False),
        scratch_types=[
            pltpu.VMEM((_CPW * _NCH, 16), jnp.int32),
            pltpu.VMEM((_CPW, _S), jnp.int32),
            pltpu.VMEM((32,), jnp.int32),
            pltpu.VMEM((16,), jnp.int32),
        ],
    )
    def k(bits_hbm, dbj_hbm, out_hbm, bits_v, out_v, tbl_v, vtmp_v):
        wid = lax.axis_index("s") * 2 + lax.axis_index("c")
        base = wid * _CPW
        boff = (base // _P) * _N
        pltpu.sync_copy(dbj_hbm, tbl_v)
        pltpu.sync_copy(bits_hbm.at[pl.ds(base * _NCH, _CPW * _NCH)], bits_v)
        lanes = lax.iota(jnp.int32, 16)
        zero16 = jnp.zeros((16,), jnp.int32)
        big = jnp.int32(2 ** 30)

        def ctz16(t):
            i = ((t * 0x077CB531) >> 27) & 31
            return plsc.load_gather(tbl_v, [i])

        def col_body(col, carry):
            colv = zero16 + col

            def chunk_cond(st):
                ch, count, first = st
                return (ch < _NCH) & (count < _S)

            def chunk_body(st):
                ch, count, first = st
                v = bits_v[col * _NCH + ch]
                p1 = v - ((v >> 1) & 0x55555555)
                p2 = (p1 & 0x33333333) + ((p1 >> 2) & 0x33333333)
                p3 = (p2 + (p2 >> 4)) & 0x0F0F0F0F
                wp = ((p3 * 0x01010101) >> 24) & 63
                baser = count + plsc.cumsum(wp) - wp
                posbase = ch * 512 + lanes * 32 + boff
                fc = jnp.min(jnp.where(v != 0, ctz16(v & (-v)) + posbase, big))
                first = lax.select((count == 0) & (fc < big), fc, first)

                need = jnp.maximum(jnp.int32(_S) - baser, 0)
                iters = jnp.max(jnp.minimum(wp, need))
                vtmp_v[...] = v

                def in_body(kk, uu):
                    v2 = vtmp_v[...]
                    t = v2 & (-v2)
                    pos = ctz16(t) + posbase
                    slot = baser + kk
                    valid = (t != 0) & (slot < _S)
                    plsc.store_scatter(out_v, [colv, slot], pos, mask=valid)
                    vtmp_v[...] = v2 ^ t
                    return uu

                lax.fori_loop(0, iters, in_body, 0)
                return (ch + 1, count + jnp.sum(wp), first)

            _, count, first = lax.while_loop(
                chunk_cond, chunk_body, (jnp.int32(0), jnp.int32(0), boff))
            fillv = zero16 + first
            for g in range(2):
                sl = lanes + g * 16
                plsc.store_scatter(out_v, [colv, sl], fillv, mask=sl >= count)
            return carry

        lax.fori_loop(0, _CPW, col_body, 0)
        pltpu.sync_copy(out_v, out_hbm.at[pl.ds(base, _CPW)])

    return k(bits_rows, dbj)


_ROWS = _B * _S * _P
_RPW = _ROWS // _NW
_GCH = 128
_HALF = 2048


def _gather(table, idxflat):
    mesh = plsc.VectorSubcoreMesh(core_axis_name="c", subcore_axis_name="s")

    @functools.partial(
        pl.kernel,
        out_type=jax.ShapeDtypeStruct((_ROWS, 32), jnp.float32),
        mesh=mesh,
        compiler_params=pltpu.CompilerParams(use_tc_tiling_on_sc=False, needs_layout_passes=False),
        scratch_types=[
            pltpu.VMEM((_RPW,), jnp.int32),
            pltpu.VMEM((_HALF, 32), jnp.float32),
            pltpu.SemaphoreType.DMA,
        ],
    )
    def k(table_hbm, idx_hbm, out_hbm, idx_v, rows_v, sem):
        wid = lax.axis_index("s") * 2 + lax.axis_index("c")
        base = wid * _RPW
        pltpu.sync_copy(idx_hbm.at[pl.ds(base, _RPW)], idx_v)
        for h in range(_RPW // _HALF):
            copies = []
            for c in range(_HALF // _GCH):
                off = h * _HALF + c * _GCH
                copies.append(pltpu.async_copy(
                    table_hbm.at[idx_v.at[pl.ds(off, _GCH)]],
                    rows_v.at[pl.ds(c * _GCH, _GCH)], sem))
            for cp in copies:
                cp.wait()
            pltpu.sync_copy(rows_v, out_hbm.at[pl.ds(base + h * _HALF, _HALF)])

    return k(table, idxflat)


def _mlp_body(g_ref, nx_ref, ny_ref, nz_ref, cnt_ref,
              w1_ref, b1_ref, w2_ref, b2_ref, w3_ref, b3_ref, out_ref):
    g = g_ref[0]
    nx = nx_ref[0]
    ny = ny_ref[0]
    nz = nz_ref[0]
    cen = jnp.concatenate(
        [nx, ny, nz, jnp.zeros((_P, 29), jnp.float32)], axis=1)
    x = (g.reshape(_P, _S, 32) - cen[:, None, :]).reshape(_S * _P, 32)
    h = jnp.maximum(
        jnp.dot(x, w1_ref[...], preferred_element_type=jnp.float32)
        + b1_ref[...], 0.0)
    h = jnp.maximum(
        jnp.dot(h, w2_ref[...], preferred_element_type=jnp.float32)
        + b2_ref[...], 0.0)
    h = jnp.maximum(
        jnp.dot(h, w3_ref[...], preferred_element_type=jnp.float32)
        + b3_ref[...], 0.0)
    red = jnp.max(h.reshape(_P, _S, 64), axis=1)
    mask = cnt_ref[0] > 0
    out_ref[0] = jnp.where(mask, red, 0.0)


def _mlp_pool(g, nx, ny, nz, cnt, w1p, b1, w2t, b2, w3t, b3):
    full = lambda shape: pl.BlockSpec(shape, lambda b: (0,) * len(shape))
    return pl.pallas_call(
        _mlp_body,
        grid=(_B,),
        in_specs=[
            pl.BlockSpec((1, _S * _P, 32), lambda b: (b, 0, 0)),
            pl.BlockSpec((1, _P, 1), lambda b: (b, 0, 0)),
            pl.BlockSpec((1, _P, 1), lambda b: (b, 0, 0)),
            pl.BlockSpec((1, _P, 1), lambda b: (b, 0, 0)),
            pl.BlockSpec((1, _P, 1), lambda b: (b, 0, 0)),
            full((32, 32)), full((1, 32)),
            full((32, 32)), full((1, 32)),
            full((32, 64)), full((1, 64)),
        ],
        out_specs=pl.BlockSpec((1, _P, 64), lambda b: (b, 0, 0)),
        out_shape=jax.ShapeDtypeStruct((_B, _P, 64), jnp.float32),
    )(g, nx, ny, nz, cnt, w1p, b1, w2t, b2, w3t, b3)


def kernel(xyz, features, W1, b1, W2, b2, W3, b3):
    xs = xyz[:, :, 0]
    ys = xyz[:, :, 1]
    zs = xyz[:, :, 2]

    nx, ny, nz = _fps(xs, ys, zs)

    bits, cntg = _ballquery(
        xs[:, :, None], ys[:, :, None], zs[:, :, None],
        nx.reshape(_B, _NJ, 1, _PCH),
        ny.reshape(_B, _NJ, 1, _PCH),
        nz.reshape(_B, _NJ, 1, _PCH),
    )

    bits_rows = bits.transpose(0, 1, 3, 2).reshape(_B * _P * _NCH, 16)
    idxflat = _extract_sc(bits_rows, jnp.asarray(_DBJ)).reshape(_ROWS)
    cnt = cntg.reshape(_B, _P, 1)

    table = jnp.concatenate(
        [xyz, features.transpose(0, 2, 1),
         jnp.zeros((_B, _N, 32 - 3 - _C), jnp.float32)], axis=-1
    ).reshape(_B * _N, 32)

    g = _gather(table, idxflat).reshape(_B, _S * _P, 32)

    w1p = jnp.zeros((32, 32), jnp.float32).at[:3 + _C, :].set(W1.T)
    feats = _mlp_pool(
        g, nx[:, :, None], ny[:, :, None], nz[:, :, None], cnt,
        w1p, b1[None, :], W2.T, b2[None, :], W3.T, b3[None, :])

    new_xyz = jnp.stack([nx, ny, nz], axis=-1)
    new_features = feats.transpose(0, 2, 1)
    return (new_xyz, new_features)
